# baseline (speedup 1.0000x reference)
import functools

import jax
import jax.numpy as jnp
from jax import lax
from jax.experimental import pallas as pl
from jax.experimental.pallas import tpu as pltpu
from jax.experimental.pallas import tpu_sc as plsc

B = 4096
D_DENSE = 13
F = 26
V = 100000
E = 64
D0 = E + F * E


_NC = 2
_NS = 16
_NW = _NC * _NS
_TROWS = F * E
_RPW = _TROWS // _NW


_H = 50048
_SOFF = (0, 49920)
_TOFF = 99968


def _sc_gather_t(tswap, tail, idxT):
    mesh = plsc.VectorSubcoreMesh(core_axis_name="c", subcore_axis_name="s")

    @functools.partial(
        pl.kernel,
        mesh=mesh,
        compiler_params=pltpu.CompilerParams(use_tc_tiling_on_sc=True,
                                             needs_layout_passes=False),
        out_type=jax.ShapeDtypeStruct((2, _TROWS, B), jnp.float32),
        scratch_types=[
            pltpu.VMEM((_H,), jnp.float32),
            pltpu.VMEM((_H,), jnp.float32),
            pltpu.VMEM((32,), jnp.float32),
            pltpu.VMEM((B,), jnp.int32),
            pltpu.VMEM((B,), jnp.float32),
            pltpu.SemaphoreType.DMA,
            pltpu.SemaphoreType.DMA,
        ],
    )
    def gather_kernel(tbl, tail_hbm, idx_hbm, out_hbm, b0, b1, bt, idx_v,
                      out_v, sem0, sem1):
        wid = lax.axis_index("s") * _NC + lax.axis_index("c")
        base = wid * _RPW

        def row_ref(k):
            rid = base + k
            return tbl.at[rid // E, rid % E]

        def seg_src(k, s):
            return row_ref(k).at[pl.ds(_SOFF[s], _H)]

        def gather_seg0(k):
            for j in range(B // 16):
                sl = pl.ds(j * 16, 16)
                ii = idx_v[sl]
                m = ii < _H
                g = plsc.load_gather(b0, [jnp.where(m, ii, 0)], mask=m)
                out_v[sl] = jnp.where(m, g, 0.0)
            pltpu.sync_copy(out_v, out_hbm.at[0, base + k])

        def gather_seg1(k):
            for j in range(B // 16):
                sl = pl.ds(j * 16, 16)
                ii = idx_v[sl]
                m1 = (ii >= _H) & (ii < _TOFF)
                g1 = plsc.load_gather(b1, [jnp.where(m1, ii - _SOFF[1], 0)],
                                      mask=m1)
                m2 = ii >= _TOFF
                g2 = plsc.load_gather(bt, [jnp.where(m2, ii - _TOFF, 0)],
                                      mask=m2)
                out_v[sl] = jnp.where(m1, g1, 0.0) + jnp.where(m2, g2, 0.0)
            pltpu.sync_copy(out_v, out_hbm.at[1, base + k])

        pltpu.async_copy(seg_src(0, 0), b0, sem0)

        def body(k, carry):
            pltpu.async_copy(seg_src(k, 1), b1, sem1)
            pltpu.sync_copy(idx_hbm.at[(base + k) // E], idx_v)
            rid = base + k
            pltpu.sync_copy(tail_hbm.at[rid // E, rid % E], bt)
            pltpu.make_async_copy(seg_src(k, 0), b0, sem0).wait()
            gather_seg0(k)

            @pl.when(k + 1 < _RPW)
            def _():
                pltpu.async_copy(seg_src(k + 1, 0), b0, sem0)

            pltpu.make_async_copy(seg_src(k, 1), b1, sem1).wait()
            gather_seg1(k)
            return carry

        lax.fori_loop(0, _RPW, body, 0)

    return gather_kernel(tswap, tail, idxT)


_TILE = 256


def _mmT(w, x):
    return lax.dot_general(w, x, (((0,), (0,)), ((), ())),
                           preferred_element_type=jnp.float32)


def _dense_body_t(dT_ref, embT_ref,
                  bw0, bb0, bw1, bb1, bw2, bb2,
                  V0, U0, c0, V1, U1, c1, V2, U2, c2,
                  tw0, tb0, tw1, tb1, tw2, tb2, tw3, tb3, tw4, tb4,
                  outT_ref):
    h = jnp.maximum(_mmT(bw0[...], dT_ref[...]) + bb0[...], 0.0)
    h = jnp.maximum(_mmT(bw1[...], h) + bb1[...], 0.0)
    dh = jnp.maximum(_mmT(bw2[...], h) + bb2[...], 0.0)

    embT = embT_ref[0] + embT_ref[1]
    x0 = jnp.concatenate([dh, embT], axis=0)
    xl = x0
    for Vm, Um, cb in ((V0, U0, c0), (V1, U1, c1), (V2, U2, c2)):
        proj = _mmT(Vm[...], xl)
        u = _mmT(Um[...], proj) + cb[...]
        xl = x0 * u + xl

    h = jnp.maximum(_mmT(tw0[...], xl) + tb0[...], 0.0)
    h = jnp.maximum(_mmT(tw1[...], h) + tb1[...], 0.0)
    h = jnp.maximum(_mmT(tw2[...], h) + tb2[...], 0.0)
    h = jnp.maximum(_mmT(tw3[...], h) + tb3[...], 0.0)
    z = _mmT(tw4[...], h) + tb4[...]
    outT_ref[...] = 1.0 / (1.0 + jnp.exp(-z))


def _const_spec(shape):
    nd = len(shape)
    return pl.BlockSpec(shape, lambda i: (0,) * nd)


def _dense_chain_t(dT, embT, weights):
    grid = (B // _TILE,)
    in_specs = [
        pl.BlockSpec((D_DENSE, _TILE), lambda i: (0, i)),
        pl.BlockSpec((2, _TROWS, _TILE), lambda i: (0, 0, i)),
    ] + [_const_spec(w.shape) for w in weights]

    return pl.pallas_call(
        _dense_body_t,
        grid=grid,
        in_specs=in_specs,
        out_specs=pl.BlockSpec((1, _TILE), lambda i: (0, i)),
        out_shape=jax.ShapeDtypeStruct((1, B), jnp.float32),
        compiler_params=pltpu.CompilerParams(
            vmem_limit_bytes=100 * 1024 * 1024),
    )(dT, embT, *weights)


def kernel(dense_features, sparse_idx, emb_tables,
           bw0, bb0, bw1, bb1, bw2, bb2,
           V0, U0, c0, V1, U1, c1, V2, U2, c2,
           tw0, tb0, tw1, tb1, tw2, tb2, tw3, tb3, tw4, tb4):
    tswap = jnp.swapaxes(emb_tables, 1, 2)
    tail = jnp.swapaxes(emb_tables[:, _TOFF:, :], 1, 2)
    idxT = sparse_idx.T
    embT = _sc_gather_t(tswap, tail, idxT)

    weights = (bw0, bb0.reshape(-1, 1), bw1, bb1.reshape(-1, 1),
               bw2, bb2.reshape(-1, 1),
               V0, U0, c0.reshape(-1, 1), V1, U1, c1.reshape(-1, 1),
               V2, U2, c2.reshape(-1, 1),
               tw0, tb0.reshape(-1, 1), tw1, tb1.reshape(-1, 1),
               tw2, tb2.reshape(-1, 1), tw3, tb3.reshape(-1, 1),
               tw4, tb4.reshape(-1, 1))
    outT = _dense_chain_t(dense_features.T, embT, weights)
    return outT.reshape(B, 1)


# speedup vs baseline: 1.1841x; 1.1841x over previous
import functools

import jax
import jax.numpy as jnp
from jax import lax
from jax.experimental import pallas as pl
from jax.experimental.pallas import tpu as pltpu
from jax.experimental.pallas import tpu_sc as plsc

B = 4096
D_DENSE = 13
F = 26
V = 100000
E = 64
D0 = E + F * E


_NC = 2
_NS = 16
_NW = _NC * _NS
_TROWS = F * E
_RPW = _TROWS // _NW


_H = 50048
_SOFF = (0, 49920)
_TOFF = 99968


def _sc_gather_t(tswap, tail, idxT):
    mesh = plsc.VectorSubcoreMesh(core_axis_name="c", subcore_axis_name="s")

    @functools.partial(
        pl.kernel,
        mesh=mesh,
        compiler_params=pltpu.CompilerParams(use_tc_tiling_on_sc=True,
                                             needs_layout_passes=False),
        out_type=jax.ShapeDtypeStruct((_TROWS, B), jnp.float32),
        scratch_types=[
            pltpu.VMEM((_H,), jnp.float32),
            pltpu.VMEM((_H,), jnp.float32),
            pltpu.VMEM((E * 32,), jnp.float32),
            pltpu.VMEM((B,), jnp.int32),
            pltpu.VMEM((B,), jnp.float32),
            pltpu.SemaphoreType.DMA,
            pltpu.SemaphoreType.DMA,
        ],
    )
    def gather_kernel(tbl, tailsq, idx_hbm, out_hbm, b0, b1, bt, idx_v,
                      out_v, sem0, sem1):
        wid = lax.axis_index("s") * _NC + lax.axis_index("c")
        base = wid * _RPW

        def seg_src(k, s):
            rid = base + k
            return tbl.at[rid // E, rid % E].at[pl.ds(_SOFF[s], _H)]

        def gather_row(k):
            rid = base + k
            e = rid % E
            for j in range(B // 16):
                sl = pl.ds(j * 16, 16)
                ii = idx_v[sl]
                m0 = ii < _H
                g0 = plsc.load_gather(b0, [jnp.where(m0, ii, 0)], mask=m0)
                out_v[sl] = jnp.where(m0, g0, 0.0)
            pltpu.make_async_copy(seg_src(k, 1), b1, sem1).wait()

            @pl.when(k + 1 < _RPW)
            def _():
                pltpu.async_copy(seg_src(k + 1, 0), b0, sem0)

            for j in range(B // 16):
                sl = pl.ds(j * 16, 16)
                ii = idx_v[sl]
                m1 = (ii >= _H) & (ii < _TOFF)
                g1 = plsc.load_gather(b1, [jnp.where(m1, ii - _SOFF[1], 0)],
                                      mask=m1)
                m2 = ii >= _TOFF
                it = jnp.where(m2, e * 32 + ii - _TOFF, 0)
                g2 = plsc.load_gather(bt, [it], mask=m2)
                out_v[sl] = (out_v[sl] + jnp.where(m1, g1, 0.0)
                             + jnp.where(m2, g2, 0.0))
            pltpu.sync_copy(out_v, out_hbm.at[rid])

        pltpu.async_copy(seg_src(0, 0), b0, sem0)
        pltpu.sync_copy(tailsq.at[base // E], bt)
        pltpu.sync_copy(idx_hbm.at[base // E], idx_v)

        def body(k, carry):
            pltpu.async_copy(seg_src(k, 1), b1, sem1)
            rid = base + k

            @pl.when((rid % E == 0) & (k > 0))
            def _():
                pltpu.sync_copy(idx_hbm.at[rid // E], idx_v)
                pltpu.sync_copy(tailsq.at[rid // E], bt)

            pltpu.make_async_copy(seg_src(k, 0), b0, sem0).wait()
            gather_row(k)
            return carry

        lax.fori_loop(0, _RPW, body, 0)

    return gather_kernel(tswap, tail, idxT)


_TILE = 256


def _mmT(w, x):
    return lax.dot_general(w, x, (((0,), (0,)), ((), ())),
                           preferred_element_type=jnp.float32)


def _dense_body_t(dT_ref, embT_ref,
                  bw0, bb0, bw1, bb1, bw2, bb2,
                  V0, U0, c0, V1, U1, c1, V2, U2, c2,
                  tw0, tb0, tw1, tb1, tw2, tb2, tw3, tb3, tw4, tb4,
                  outT_ref):
    h = jnp.maximum(_mmT(bw0[...], dT_ref[...]) + bb0[...], 0.0)
    h = jnp.maximum(_mmT(bw1[...], h) + bb1[...], 0.0)
    dh = jnp.maximum(_mmT(bw2[...], h) + bb2[...], 0.0)

    x0 = jnp.concatenate([dh, embT_ref[...]], axis=0)
    xl = x0
    for Vm, Um, cb in ((V0, U0, c0), (V1, U1, c1), (V2, U2, c2)):
        proj = _mmT(Vm[...], xl)
        u = _mmT(Um[...], proj) + cb[...]
        xl = x0 * u + xl

    h = jnp.maximum(_mmT(tw0[...], xl) + tb0[...], 0.0)
    h = jnp.maximum(_mmT(tw1[...], h) + tb1[...], 0.0)
    h = jnp.maximum(_mmT(tw2[...], h) + tb2[...], 0.0)
    h = jnp.maximum(_mmT(tw3[...], h) + tb3[...], 0.0)
    z = _mmT(tw4[...], h) + tb4[...]
    outT_ref[...] = 1.0 / (1.0 + jnp.exp(-z))


def _const_spec(shape):
    nd = len(shape)
    return pl.BlockSpec(shape, lambda i: (0,) * nd)


def _dense_chain_t(dT, embT, weights):
    grid = (B // _TILE,)
    in_specs = [
        pl.BlockSpec((D_DENSE, _TILE), lambda i: (0, i)),
        pl.BlockSpec((_TROWS, _TILE), lambda i: (0, i)),
    ] + [_const_spec(w.shape) for w in weights]

    return pl.pallas_call(
        _dense_body_t,
        grid=grid,
        in_specs=in_specs,
        out_specs=pl.BlockSpec((1, _TILE), lambda i: (0, i)),
        out_shape=jax.ShapeDtypeStruct((1, B), jnp.float32),
        compiler_params=pltpu.CompilerParams(
            vmem_limit_bytes=100 * 1024 * 1024),
    )(dT, embT, *weights)


def kernel(dense_features, sparse_idx, emb_tables,
           bw0, bb0, bw1, bb1, bw2, bb2,
           V0, U0, c0, V1, U1, c1, V2, U2, c2,
           tw0, tb0, tw1, tb1, tw2, tb2, tw3, tb3, tw4, tb4):
    tswap = jnp.swapaxes(emb_tables, 1, 2)
    tail = jnp.swapaxes(emb_tables[:, _TOFF:, :], 1, 2).reshape(F, E * 32)
    idxT = sparse_idx.T
    embT = _sc_gather_t(tswap, tail, idxT)

    weights = (bw0, bb0.reshape(-1, 1), bw1, bb1.reshape(-1, 1),
               bw2, bb2.reshape(-1, 1),
               V0, U0, c0.reshape(-1, 1), V1, U1, c1.reshape(-1, 1),
               V2, U2, c2.reshape(-1, 1),
               tw0, tb0.reshape(-1, 1), tw1, tb1.reshape(-1, 1),
               tw2, tb2.reshape(-1, 1), tw3, tb3.reshape(-1, 1),
               tw4, tb4.reshape(-1, 1))
    outT = _dense_chain_t(dense_features.T, embT, weights)
    return outT.reshape(B, 1)


# speedup vs baseline: 1.2802x; 1.0812x over previous
import functools

import jax
import jax.numpy as jnp
from jax import lax
from jax.experimental import pallas as pl
from jax.experimental.pallas import tpu as pltpu
from jax.experimental.pallas import tpu_sc as plsc

B = 4096
D_DENSE = 13
F = 26
V = 100000
E = 64
D0 = E + F * E


_NC = 2
_NS = 16
_NW = _NC * _NS
_TROWS = F * E
_RPW = _TROWS // _NW


def _sc_gather_t(tswap, idxT):
    mesh = plsc.VectorSubcoreMesh(core_axis_name="c", subcore_axis_name="s")

    @functools.partial(
        pl.kernel,
        mesh=mesh,
        compiler_params=pltpu.CompilerParams(use_tc_tiling_on_sc=True,
                                             needs_layout_passes=False),
        out_type=jax.ShapeDtypeStruct((_TROWS, B), jnp.float32),
        scratch_types=[
            pltpu.VMEM((V,), jnp.float32),
            pltpu.VMEM((B,), jnp.int32),
            pltpu.VMEM((B,), jnp.float32),
        ],
    )
    def gather_kernel(tbl, idx_hbm, out_hbm, row_v, idx_v, out_v):
        wid = lax.axis_index("s") * _NC + lax.axis_index("c")

        def body(k, carry):
            rid = wid * _RPW + k
            f = rid // E
            e = rid % E
            pltpu.sync_copy(idx_hbm.at[f], idx_v)
            pltpu.sync_copy(tbl.at[f, e], row_v)
            for j in range(B // 16):
                ii = idx_v[pl.ds(j * 16, 16)]
                out_v[pl.ds(j * 16, 16)] = plsc.load_gather(row_v, [ii])
            pltpu.sync_copy(out_v, out_hbm.at[rid])
            return carry

        lax.fori_loop(0, _RPW, body, 0)

    return gather_kernel(tswap, idxT)


_TILE = 256


def _mmT(w, x):
    return lax.dot_general(w, x.astype(jnp.bfloat16), (((0,), (0,)), ((), ())),
                           preferred_element_type=jnp.float32)


def _dense_body_t(dT_ref, embT_ref,
                  bw0, bb0, bw1, bb1, bw2, bb2,
                  V0, U0, c0, V1, U1, c1, V2, U2, c2,
                  tw0, tb0, tw1, tb1, tw2, tb2, tw3, tb3, tw4, tb4,
                  outT_ref):
    h = jnp.maximum(_mmT(bw0[...], dT_ref[...]) + bb0[...], 0.0)
    h = jnp.maximum(_mmT(bw1[...], h) + bb1[...], 0.0)
    dh = jnp.maximum(_mmT(bw2[...], h) + bb2[...], 0.0)

    x0 = jnp.concatenate([dh, embT_ref[...]], axis=0)
    xl = x0
    for Vm, Um, cb in ((V0, U0, c0), (V1, U1, c1), (V2, U2, c2)):
        proj = _mmT(Vm[...], xl)
        u = _mmT(Um[...], proj) + cb[...]
        xl = x0 * u + xl

    h = jnp.maximum(_mmT(tw0[...], xl) + tb0[...], 0.0)
    h = jnp.maximum(_mmT(tw1[...], h) + tb1[...], 0.0)
    h = jnp.maximum(_mmT(tw2[...], h) + tb2[...], 0.0)
    h = jnp.maximum(_mmT(tw3[...], h) + tb3[...], 0.0)
    z = _mmT(tw4[...], h) + tb4[...]
    outT_ref[...] = 1.0 / (1.0 + jnp.exp(-z))


def _const_spec(shape):
    nd = len(shape)
    return pl.BlockSpec(shape, lambda i: (0,) * nd)


def _dense_chain_t(dT, embT, weights):
    grid = (B // _TILE,)
    in_specs = [
        pl.BlockSpec((D_DENSE, _TILE), lambda i: (0, i)),
        pl.BlockSpec((_TROWS, _TILE), lambda i: (0, i)),
    ] + [_const_spec(w.shape) for w in weights]

    return pl.pallas_call(
        _dense_body_t,
        grid=grid,
        in_specs=in_specs,
        out_specs=pl.BlockSpec((1, _TILE), lambda i: (0, i)),
        out_shape=jax.ShapeDtypeStruct((1, B), jnp.float32),
        compiler_params=pltpu.CompilerParams(
            vmem_limit_bytes=100 * 1024 * 1024),
    )(dT, embT, *weights)


def kernel(dense_features, sparse_idx, emb_tables,
           bw0, bb0, bw1, bb1, bw2, bb2,
           V0, U0, c0, V1, U1, c1, V2, U2, c2,
           tw0, tb0, tw1, tb1, tw2, tb2, tw3, tb3, tw4, tb4):
    tswap = jnp.swapaxes(emb_tables, 1, 2)
    idxT = sparse_idx.T
    embT = _sc_gather_t(tswap, idxT)

    bf = lambda w: w.astype(jnp.bfloat16)
    weights = (bf(bw0), bb0.reshape(-1, 1), bf(bw1), bb1.reshape(-1, 1),
               bf(bw2), bb2.reshape(-1, 1),
               bf(V0), bf(U0), c0.reshape(-1, 1),
               bf(V1), bf(U1), c1.reshape(-1, 1),
               bf(V2), bf(U2), c2.reshape(-1, 1),
               bf(tw0), tb0.reshape(-1, 1), bf(tw1), tb1.reshape(-1, 1),
               bf(tw2), tb2.reshape(-1, 1), bf(tw3), tb3.reshape(-1, 1),
               bf(tw4), tb4.reshape(-1, 1))
    outT = _dense_chain_t(dense_features.T, embT, weights)
    return outT.reshape(B, 1)


# speedup vs baseline: 1.6590x; 1.2959x over previous
import functools

import jax
import jax.numpy as jnp
from jax import lax
from jax.experimental import pallas as pl
from jax.experimental.pallas import tpu as pltpu
from jax.experimental.pallas import tpu_sc as plsc

B = 4096
D_DENSE = 13
F = 26
V = 100000
E = 64
D0 = E + F * E


_NC = 2
_NS = 16
_NW = _NC * _NS
_TROWS = F * E
_RPW = _TROWS // _NW


def _sc_gather_t(tswap, idxT):
    mesh = plsc.VectorSubcoreMesh(core_axis_name="c", subcore_axis_name="s")

    @functools.partial(
        pl.kernel,
        mesh=mesh,
        compiler_params=pltpu.CompilerParams(use_tc_tiling_on_sc=True,
                                             needs_layout_passes=False),
        out_type=jax.ShapeDtypeStruct((_TROWS, B), jnp.float32),
        scratch_types=[
            pltpu.VMEM((V,), jnp.float32),
            pltpu.VMEM((B,), jnp.int32),
            pltpu.VMEM((B,), jnp.float32),
            pltpu.VMEM((B,), jnp.float32),
            pltpu.SemaphoreType.DMA,
            pltpu.SemaphoreType.DMA,
        ],
    )
    def gather_kernel(tbl, idx_hbm, out_hbm, row_v, idx_v, out_a, out_b,
                      sem_a, sem_b):
        wid = lax.axis_index("s") * _NC + lax.axis_index("c")
        base = wid * _RPW

        def body(k2, carry):
            for half, (ov, sem) in enumerate(((out_a, sem_a), (out_b, sem_b))):
                k = 2 * k2 + half
                rid = base + k
                f = rid // E
                e = rid % E

                if half == 0:
                    @pl.when((e == 0) | (k2 == 0))
                    def _():
                        pltpu.sync_copy(idx_hbm.at[f], idx_v)
                else:
                    @pl.when(e == 0)
                    def _():
                        pltpu.sync_copy(idx_hbm.at[f], idx_v)

                pltpu.sync_copy(tbl.at[f, e], row_v)

                @pl.when(k2 > 0)
                def _():
                    pltpu.make_async_copy(ov, out_hbm.at[rid], sem).wait()

                for j in range(B // 16):
                    ii = idx_v[pl.ds(j * 16, 16)]
                    ov[pl.ds(j * 16, 16)] = plsc.load_gather(row_v, [ii])
                pltpu.async_copy(ov, out_hbm.at[rid], sem)
            return carry

        lax.fori_loop(0, _RPW // 2, body, 0)
        pltpu.make_async_copy(out_a, out_hbm.at[base], sem_a).wait()
        pltpu.make_async_copy(out_b, out_hbm.at[base], sem_b).wait()

    return gather_kernel(tswap, idxT)


_TILE = 1024


def _mmT(w, x):
    return lax.dot_general(w, x.astype(jnp.bfloat16), (((0,), (0,)), ((), ())),
                           preferred_element_type=jnp.float32)


def _dense_body_t(dT_ref, embT_ref,
                  bw0, bb0, bw1, bb1, bw2, bb2,
                  V0, U0, c0, V1, U1, c1, V2, U2, c2,
                  tw0, tb0, tw1, tb1, tw2, tb2, tw3, tb3, tw4, tb4,
                  outT_ref):
    h = jnp.maximum(_mmT(bw0[...], dT_ref[...]) + bb0[...], 0.0)
    h = jnp.maximum(_mmT(bw1[...], h) + bb1[...], 0.0)
    dh = jnp.maximum(_mmT(bw2[...], h) + bb2[...], 0.0)

    x0 = jnp.concatenate([dh, embT_ref[...]], axis=0)
    xl = x0
    for Vm, Um, cb in ((V0, U0, c0), (V1, U1, c1), (V2, U2, c2)):
        proj = _mmT(Vm[...], xl)
        u = _mmT(Um[...], proj) + cb[...]
        xl = x0 * u + xl

    h = jnp.maximum(_mmT(tw0[...], xl) + tb0[...], 0.0)
    h = jnp.maximum(_mmT(tw1[...], h) + tb1[...], 0.0)
    h = jnp.maximum(_mmT(tw2[...], h) + tb2[...], 0.0)
    h = jnp.maximum(_mmT(tw3[...], h) + tb3[...], 0.0)
    z = _mmT(tw4[...], h) + tb4[...]
    outT_ref[...] = 1.0 / (1.0 + jnp.exp(-z))


def _const_spec(shape):
    nd = len(shape)
    return pl.BlockSpec(shape, lambda i: (0,) * nd)


def _dense_chain_t(dT, embT, weights):
    grid = (B // _TILE,)
    in_specs = [
        pl.BlockSpec((D_DENSE, _TILE), lambda i: (0, i)),
        pl.BlockSpec((_TROWS, _TILE), lambda i: (0, i)),
    ] + [_const_spec(w.shape) for w in weights]

    return pl.pallas_call(
        _dense_body_t,
        grid=grid,
        in_specs=in_specs,
        out_specs=pl.BlockSpec((1, _TILE), lambda i: (0, i)),
        out_shape=jax.ShapeDtypeStruct((1, B), jnp.float32),
        compiler_params=pltpu.CompilerParams(
            vmem_limit_bytes=100 * 1024 * 1024),
    )(dT, embT, *weights)


def kernel(dense_features, sparse_idx, emb_tables,
           bw0, bb0, bw1, bb1, bw2, bb2,
           V0, U0, c0, V1, U1, c1, V2, U2, c2,
           tw0, tb0, tw1, tb1, tw2, tb2, tw3, tb3, tw4, tb4):
    tswap = jnp.swapaxes(emb_tables, 1, 2)
    idxT = sparse_idx.T
    embT = _sc_gather_t(tswap, idxT)

    bf = lambda w: w.astype(jnp.bfloat16)
    weights = (bf(bw0), bb0.reshape(-1, 1), bf(bw1), bb1.reshape(-1, 1),
               bf(bw2), bb2.reshape(-1, 1),
               bf(V0), bf(U0), c0.reshape(-1, 1),
               bf(V1), bf(U1), c1.reshape(-1, 1),
               bf(V2), bf(U2), c2.reshape(-1, 1),
               bf(tw0), tb0.reshape(-1, 1), bf(tw1), tb1.reshape(-1, 1),
               bf(tw2), tb2.reshape(-1, 1), bf(tw3), tb3.reshape(-1, 1),
               bf(tw4), tb4.reshape(-1, 1))
    outT = _dense_chain_t(dense_features.T, embT, weights)
    return outT.reshape(B, 1)
